# baseline (speedup 1.0000x reference)
import functools

import jax
import jax.numpy as jnp
from jax import lax
from jax.experimental import pallas as pl
from jax.experimental.pallas import tpu as pltpu
from jax.experimental.pallas import tpu_sc as plsc

_NUM_CORES = 2
_NUM_SUBCORES = 16
_NUM_WORKERS = _NUM_CORES * _NUM_SUBCORES
_LANES = 16

_BT = 256


def _gate_body(x_ref, gw_ref, out_ref):
    out_ref[...] = lax.dot_general(
        gw_ref[...], x_ref[...], (((1,), (1,)), ((), ())),
        preferred_element_type=jnp.float32)


def _make_router(T, E):
    chunk = T // _NUM_WORKERS
    mesh = plsc.VectorSubcoreMesh(core_axis_name="c", subcore_axis_name="s")

    @functools.partial(
        pl.kernel,
        mesh=mesh,
        out_type=(
            jax.ShapeDtypeStruct((T,), jnp.int32),
            jax.ShapeDtypeStruct((T,), jnp.int32),
            jax.ShapeDtypeStruct((T,), jnp.float32),
            jax.ShapeDtypeStruct((T,), jnp.float32),
        ),
        scratch_types=[
            pltpu.VMEM((E, chunk), jnp.float32),
            pltpu.VMEM((chunk,), jnp.int32),
            pltpu.VMEM((chunk,), jnp.int32),
            pltpu.VMEM((chunk,), jnp.float32),
            pltpu.VMEM((chunk,), jnp.float32),
        ],
    )
    def router(logits_hbm, i0_hbm, i1_hbm, w0_hbm, w1_hbm,
               lv, i0v, i1v, w0v, w1v):
        wid = lax.axis_index("s") * _NUM_CORES + lax.axis_index("c")
        base = wid * chunk
        pltpu.sync_copy(logits_hbm.at[:, pl.ds(base, chunk)], lv)
        for g in range(chunk // _LANES):
            sl = pl.ds(g * _LANES, _LANES)
            ls = [lv[e, sl] for e in range(E)]
            best = ls[0]
            bidx = jnp.zeros((_LANES,), jnp.int32)
            sec = jnp.full((_LANES,), -jnp.inf, jnp.float32)
            sidx = jnp.zeros((_LANES,), jnp.int32)
            for e in range(1, E):
                le = ls[e]
                evec = jnp.full((_LANES,), e, jnp.int32)
                gtb = le > best
                gts = le > sec
                sec = jnp.where(gtb, best, jnp.where(gts, le, sec))
                sidx = jnp.where(gtb, bidx, jnp.where(gts, evec, sidx))
                best = jnp.where(gtb, le, best)
                bidx = jnp.where(gtb, evec, bidx)
            z = jnp.full((_LANES,), 0.0, jnp.float32)
            for e in range(E):
                z = z + jnp.exp(ls[e] - best)
            pb = 1.0 / z
            ps = jnp.exp(sec - best) / z
            den = pb + ps + 1e-20
            i0v[sl] = bidx
            i1v[sl] = sidx
            w0v[sl] = pb / den
            w1v[sl] = ps / den
        pltpu.sync_copy(i0v, i0_hbm.at[pl.ds(base, chunk)])
        pltpu.sync_copy(i1v, i1_hbm.at[pl.ds(base, chunk)])
        pltpu.sync_copy(w0v, w0_hbm.at[pl.ds(base, chunk)])
        pltpu.sync_copy(w1v, w1_hbm.at[pl.ds(base, chunk)])

    return router


def _pre_body(x_ref, w1_ref, b1_ref, ws1_ref, bs1_ref, ws2_ref, bs2_ref,
              h_ref, s_ref):
    xb = x_ref[...].astype(jnp.bfloat16)
    h = jnp.maximum(
        jnp.dot(xb, w1_ref[...], preferred_element_type=jnp.float32)
        + b1_ref[...], 0.0)
    h_ref[...] = h.astype(jnp.bfloat16)
    s = jnp.maximum(
        jnp.dot(xb, ws1_ref[...], preferred_element_type=jnp.float32)
        + bs1_ref[...], 0.0)
    s = jnp.dot(s.astype(jnp.bfloat16), ws2_ref[...],
                preferred_element_type=jnp.float32) + bs2_ref[...]
    s_ref[...] = s.astype(jnp.bfloat16)


def _post_body(E, H, h_ref, s_ref, w2_ref, b2_ref,
               i0_ref, i1_ref, w0_ref, w1w_ref, out_ref):
    i0 = i0_ref[...]
    i1 = i1_ref[...]
    w0 = w0_ref[...]
    w1w = w1w_ref[...]
    eidx = lax.broadcasted_iota(jnp.int32, (_BT, E * H), 1) // H
    gate = (jnp.where(eidx == i0, w0, 0.0)
            + jnp.where(eidx == i1, w1w, 0.0))
    hw = (h_ref[...].astype(jnp.float32) * gate).astype(jnp.bfloat16)
    y = jnp.dot(hw, w2_ref[...], preferred_element_type=jnp.float32)
    e8 = lax.broadcasted_iota(jnp.int32, (_BT, E), 1)
    mv = (jnp.where(e8 == i0, w0, 0.0)
          + jnp.where(e8 == i1, w1w, 0.0))
    y = y + jnp.dot(mv, b2_ref[...], preferred_element_type=jnp.float32)
    out_ref[...] = y + s_ref[...].astype(jnp.float32)


def kernel(hidden_states, gate_w, w1, b1, w2, b2, ws1, bs1, ws2, bs2):
    b, s, d = hidden_states.shape
    T = b * s
    E, D, H = w1.shape
    EH = E * H
    HS = ws1.shape[1]
    x = hidden_states.reshape(T, d)

    logits_t = pl.pallas_call(
        _gate_body,
        grid=(T // _BT,),
        in_specs=[
            pl.BlockSpec((_BT, D), lambda i: (i, 0)),
            pl.BlockSpec((E, D), lambda i: (0, 0)),
        ],
        out_specs=pl.BlockSpec((E, _BT), lambda i: (0, i)),
        out_shape=jax.ShapeDtypeStruct((E, T), jnp.float32),
    )(x, gate_w)

    i0, i1, wt0, wt1 = _make_router(T, E)(logits_t)
    i0 = i0.reshape(T, 1)
    i1 = i1.reshape(T, 1)
    wt0 = wt0.reshape(T, 1)
    wt1 = wt1.reshape(T, 1)

    w1f = w1.transpose(1, 0, 2).reshape(D, EH).astype(jnp.bfloat16)
    b1f = b1.reshape(1, EH)
    h_all, s_all = pl.pallas_call(
        _pre_body,
        grid=(T // _BT,),
        in_specs=[
            pl.BlockSpec((_BT, D), lambda i: (i, 0)),
            pl.BlockSpec((D, EH), lambda i: (0, 0)),
            pl.BlockSpec((1, EH), lambda i: (0, 0)),
            pl.BlockSpec((D, HS), lambda i: (0, 0)),
            pl.BlockSpec((1, HS), lambda i: (0, 0)),
            pl.BlockSpec((HS, D), lambda i: (0, 0)),
            pl.BlockSpec((1, D), lambda i: (0, 0)),
        ],
        out_specs=(
            pl.BlockSpec((_BT, EH), lambda i: (i, 0)),
            pl.BlockSpec((_BT, D), lambda i: (i, 0)),
        ),
        out_shape=(
            jax.ShapeDtypeStruct((T, EH), jnp.bfloat16),
            jax.ShapeDtypeStruct((T, D), jnp.bfloat16),
        ),
    )(x, w1f, b1f, ws1.astype(jnp.bfloat16), bs1.reshape(1, HS),
      ws2.astype(jnp.bfloat16), bs2.reshape(1, D))

    out = pl.pallas_call(
        functools.partial(_post_body, E, H),
        grid=(T // _BT,),
        in_specs=[
            pl.BlockSpec((_BT, EH), lambda i: (i, 0)),
            pl.BlockSpec((_BT, D), lambda i: (i, 0)),
            pl.BlockSpec((EH, D), lambda i: (0, 0)),
            pl.BlockSpec((E, D), lambda i: (0, 0)),
            pl.BlockSpec((_BT, 1), lambda i: (i, 0)),
            pl.BlockSpec((_BT, 1), lambda i: (i, 0)),
            pl.BlockSpec((_BT, 1), lambda i: (i, 0)),
            pl.BlockSpec((_BT, 1), lambda i: (i, 0)),
        ],
        out_specs=pl.BlockSpec((_BT, D), lambda i: (i, 0)),
        out_shape=jax.ShapeDtypeStruct((T, D), jnp.float32),
    )(h_all, s_all, w2.reshape(EH, D).astype(jnp.bfloat16), b2,
      i0, i1, wt0, wt1)

    return out.reshape(b, s, d)


# speedup vs baseline: 1.5295x; 1.5295x over previous
import functools

import jax
import jax.numpy as jnp
from jax import lax
from jax.experimental import pallas as pl

_BT = 256


def _body(E, H, x_ref, gwt_ref, w1_ref, b1_ref, w2_ref, b2_ref,
          ws1_ref, bs1_ref, ws2_ref, bs2_ref, out_ref):
    x = x_ref[...]
    l = jnp.dot(x, gwt_ref[...], preferred_element_type=jnp.float32)
    mx = jnp.max(l, axis=1, keepdims=True)
    p = jnp.exp(l - mx)
    z = jnp.sum(p, axis=1, keepdims=True)
    i8 = lax.broadcasted_iota(jnp.int32, (_BT, E), 1)
    is1 = l == mx
    idx1 = jnp.min(jnp.where(is1, i8, E), axis=1, keepdims=True)
    lm = jnp.where(i8 == idx1, -jnp.inf, l)
    mx2 = jnp.max(lm, axis=1, keepdims=True)
    idx2 = jnp.min(jnp.where(lm == mx2, i8, E), axis=1, keepdims=True)
    p1 = 1.0 / z
    p2 = jnp.exp(mx2 - mx) / z
    den = p1 + p2 + 1e-20
    m = (jnp.where(i8 == idx1, p1 / den, 0.0)
         + jnp.where(i8 == idx2, p2 / den, 0.0))
    xb = x.astype(jnp.bfloat16)
    h = jnp.maximum(
        jnp.dot(xb, w1_ref[...], preferred_element_type=jnp.float32)
        + b1_ref[...], 0.0)
    expand = (lax.broadcasted_iota(jnp.int32, (E, E * H), 1) // H
              == lax.broadcasted_iota(jnp.int32, (E, E * H), 0)
              ).astype(jnp.float32)
    gate = jnp.dot(m, expand, preferred_element_type=jnp.float32)
    hw = (h * gate).astype(jnp.bfloat16)
    y = jnp.dot(hw, w2_ref[...], preferred_element_type=jnp.float32)
    y = y + jnp.dot(m, b2_ref[...], preferred_element_type=jnp.float32)
    s = jnp.maximum(
        jnp.dot(xb, ws1_ref[...], preferred_element_type=jnp.float32)
        + bs1_ref[...], 0.0)
    s = jnp.dot(s.astype(jnp.bfloat16), ws2_ref[...],
                preferred_element_type=jnp.float32) + bs2_ref[...]
    out_ref[...] = y + s


def kernel(hidden_states, gate_w, w1, b1, w2, b2, ws1, bs1, ws2, bs2):
    b, s, d = hidden_states.shape
    T = b * s
    E, D, H = w1.shape
    EH = E * H
    HS = ws1.shape[1]
    x = hidden_states.reshape(T, d)

    w1f = w1.transpose(1, 0, 2).reshape(D, EH).astype(jnp.bfloat16)
    b1f = b1.reshape(1, EH)
    out = pl.pallas_call(
        functools.partial(_body, E, H),
        grid=(T // _BT,),
        in_specs=[
            pl.BlockSpec((_BT, D), lambda i: (i, 0)),
            pl.BlockSpec((D, E), lambda i: (0, 0)),
            pl.BlockSpec((D, EH), lambda i: (0, 0)),
            pl.BlockSpec((1, EH), lambda i: (0, 0)),
            pl.BlockSpec((EH, D), lambda i: (0, 0)),
            pl.BlockSpec((E, D), lambda i: (0, 0)),
            pl.BlockSpec((D, HS), lambda i: (0, 0)),
            pl.BlockSpec((1, HS), lambda i: (0, 0)),
            pl.BlockSpec((HS, D), lambda i: (0, 0)),
            pl.BlockSpec((1, D), lambda i: (0, 0)),
        ],
        out_specs=pl.BlockSpec((_BT, D), lambda i: (i, 0)),
        out_shape=jax.ShapeDtypeStruct((T, D), jnp.float32),
    )(x, gate_w.T, w1f, b1f, w2.reshape(EH, D).astype(jnp.bfloat16), b2,
      ws1.astype(jnp.bfloat16), bs1.reshape(1, HS),
      ws2.astype(jnp.bfloat16), bs2.reshape(1, D))

    return out.reshape(b, s, d)
